# Initial kernel scaffold; baseline (speedup 1.0000x reference)
import jax
import jax.numpy as jnp
from jax.experimental import pallas as pl


def kernel(input, depth):
    raise NotImplementedError("write your pallas kernel here")


# speedup vs baseline: 12.2244x; 12.2244x over previous
import jax
import jax.numpy as jnp
from jax.experimental import pallas as pl
from jax.experimental.pallas import tpu as pltpu


def _bf(a):
    return a.astype(jnp.bfloat16).astype(jnp.float32)


def _shift_r(a):
    z = jnp.zeros(a.shape[:-1] + (1,), a.dtype)
    return jnp.concatenate([z, a[..., :-1]], axis=-1)


def _shift_d(a):
    z = jnp.zeros(a.shape[:-2] + (1, a.shape[-1]), a.dtype)
    return jnp.concatenate([z, a[..., :-1, :]], axis=-2)


def _parity_images(left, right):
    e = jnp.concatenate([left[..., :64], right[..., :64]], axis=-1)
    o = jnp.concatenate([left[..., 64:], right[..., 64:]], axis=-1)
    return e, o


def _deinterleave(ref):
    nd = len(ref.shape)
    full = (slice(None),) * (nd - 2)
    rows_e = ref[full + (slice(0, None, 2), slice(None))]
    rows_o = ref[full + (slice(1, None, 2), slice(None))]
    perm = jax.lax.broadcasted_iota(jnp.int32, rows_e.shape, rows_e.ndim - 1)
    perm = (perm % 64) * 2 + perm // 64
    pe = _bf(jnp.take_along_axis(rows_e, perm, axis=-1))
    po = _bf(jnp.take_along_axis(rows_o, perm, axis=-1))
    return pe, po


def _pool_body(xl_ref, xr_ref, dl_ref, dr_ref, o_ref):
    Ho = xl_ref.shape[2] // 2
    Wo = 128

    dle, dlo = _deinterleave(dl_ref.at[0, 0])
    dre, dro = _deinterleave(dr_ref.at[0, 0])
    d0, deo = _parity_images(dle, dre)
    doe, doo = _parity_images(dlo, dro)

    row_ok = jax.lax.broadcasted_iota(jnp.int32, (Ho, Wo), 0) > 0
    col_ok = jax.lax.broadcasted_iota(jnp.int32, (Ho, Wo), 1) > 0

    def w(dv):
        return jnp.exp(-jnp.abs(dv - d0))

    sr_doo = _shift_r(doo)
    w_e = w(deo)
    w_w = jnp.where(col_ok, w(_shift_r(deo)), 0.0)
    w_s = w(doe)
    w_n = jnp.where(row_ok, w(_shift_d(doe)), 0.0)
    w_se = w(doo)
    w_sw = jnp.where(col_ok, w(sr_doo), 0.0)
    w_ne = jnp.where(row_ok, w(_shift_d(doo)), 0.0)
    w_nw = jnp.where(row_ok & col_ok, w(_shift_d(sr_doo)), 0.0)

    rv = jnp.where(row_ok, 3.0, 2.0)
    cv = jnp.where(col_ok, 3.0, 2.0)
    inv_count = 1.0 / (rv * cv)

    xle, xlo = _deinterleave(xl_ref.at[0])
    xre, xro = _deinterleave(xr_ref.at[0])
    xee, xeo = _parity_images(xle, xre)
    xoe, xoo = _parity_images(xlo, xro)
    sr_xoo = _shift_r(xoo)

    num = (xee
           + w_e * xeo + w_w * _shift_r(xeo)
           + w_s * xoe + w_n * _shift_d(xoe)
           + w_se * xoo + w_sw * sr_xoo
           + w_ne * _shift_d(xoo) + w_nw * _shift_d(sr_xoo))
    o_ref[0] = num * inv_count


def kernel(input, depth):
    B, C, H, W = input.shape
    CB = 32
    grid = (B, C // CB)
    Wh = W // 2
    return pl.pallas_call(
        _pool_body,
        grid=grid,
        in_specs=[
            pl.BlockSpec((1, CB, H, Wh), lambda b, c: (b, c, 0, 0)),
            pl.BlockSpec((1, CB, H, Wh), lambda b, c: (b, c, 0, 1)),
            pl.BlockSpec((1, 1, H, Wh), lambda b, c: (b, 0, 0, 0)),
            pl.BlockSpec((1, 1, H, Wh), lambda b, c: (b, 0, 0, 1)),
        ],
        out_specs=pl.BlockSpec((1, CB, H // 2, W // 2),
                               lambda b, c: (b, c, 0, 0)),
        out_shape=jax.ShapeDtypeStruct((B, C, H // 2, W // 2), input.dtype),
        compiler_params=pltpu.CompilerParams(
            dimension_semantics=("parallel", "parallel"),
            vmem_limit_bytes=100 * 1024 * 1024,
        ),
    )(input, input, depth, depth)
